# baseline (speedup 1.0000x reference)
import functools

import jax
import jax.numpy as jnp
from jax import lax
from jax.experimental import pallas as pl
from jax.experimental.pallas import tpu as pltpu
from jax.experimental.pallas import tpu_sc as plsc

EMBEDDING_DIM = 64
BATCH = 16384

_info = plsc.get_sparse_core_info()
_NC, _NS, _NL = _info.num_cores, _info.num_subcores, _info.num_lanes
_NW = _NC * _NS
_B_PER_W = BATCH // _NW
_N_CHUNKS = _B_PER_W // _NL

_mesh = plsc.VectorSubcoreMesh(core_axis_name="c", subcore_axis_name="s")


@functools.partial(
    pl.kernel,
    mesh=_mesh,
    out_type=jax.ShapeDtypeStruct((BATCH, EMBEDDING_DIM), jnp.float32),
    compiler_params=pltpu.CompilerParams(needs_layout_passes=False),
    scratch_types=[
        pltpu.VMEM((_B_PER_W,), jnp.int32),
        pltpu.VMEM((_B_PER_W, EMBEDDING_DIM), jnp.float32),
        pltpu.SemaphoreType.DMA,
    ],
)
def _emb_lookup(idx_hbm, table_hbm, out_hbm, idx_v, rows_v, sem):
    wid = lax.axis_index("s") * _NC + lax.axis_index("c")
    base = wid * _B_PER_W
    pltpu.sync_copy(idx_hbm.at[pl.ds(base, _B_PER_W)], idx_v)

    lane = lax.iota(jnp.int32, _NL)

    def fire(k, _):
        chunk = idx_v[pl.ds(k * _NL, _NL)]
        for j in range(_NL):
            r = jnp.sum(jnp.where(lane == j, chunk, 0))
            pltpu.async_copy(table_hbm.at[r], rows_v.at[k * _NL + j], sem)
        return ()

    lax.fori_loop(0, _N_CHUNKS, fire, ())

    def drain(i, _):
        pltpu.make_async_copy(table_hbm.at[0], rows_v.at[0], sem).wait()
        return ()

    lax.fori_loop(0, _B_PER_W, drain, ())
    pltpu.sync_copy(rows_v, out_hbm.at[pl.ds(base, _B_PER_W)])


def kernel(x, table):
    return _emb_lookup(x.astype(jnp.int32), table)


# speedup vs baseline: 1.0043x; 1.0043x over previous
import functools

import jax
import jax.numpy as jnp
from jax import lax
from jax.experimental import pallas as pl
from jax.experimental.pallas import tpu as pltpu
from jax.experimental.pallas import tpu_sc as plsc

EMBEDDING_DIM = 64
BATCH = 16384
_NSEM = 4

_info = plsc.get_sparse_core_info()
_NC, _NS, _NL = _info.num_cores, _info.num_subcores, _info.num_lanes
_NW = _NC * _NS
_B_PER_W = BATCH // _NW
_N_CHUNKS = _B_PER_W // _NL

_mesh = plsc.VectorSubcoreMesh(core_axis_name="c", subcore_axis_name="s")


@functools.partial(
    pl.kernel,
    mesh=_mesh,
    out_type=jax.ShapeDtypeStruct((BATCH, EMBEDDING_DIM), jnp.float32),
    compiler_params=pltpu.CompilerParams(needs_layout_passes=False),
    scratch_types=[
        pltpu.VMEM((_B_PER_W,), jnp.int32),
        pltpu.VMEM((_B_PER_W, EMBEDDING_DIM), jnp.float32),
    ]
    + [pltpu.SemaphoreType.DMA] * _NSEM,
)
def _emb_lookup(idx_hbm, table_hbm, out_hbm, idx_v, rows_v, *sems):
    wid = lax.axis_index("s") * _NC + lax.axis_index("c")
    base = wid * _B_PER_W
    pltpu.sync_copy(idx_hbm.at[pl.ds(base, _B_PER_W)], idx_v)

    lane = lax.iota(jnp.int32, _NL)

    def fire(k, _):
        chunk = idx_v[pl.ds(k * _NL, _NL)]
        for j in range(_NL):
            r = jnp.sum(jnp.where(lane == j, chunk, 0))
            pltpu.async_copy(
                table_hbm.at[r], rows_v.at[k * _NL + j], sems[j % _NSEM]
            )
        return ()

    lax.fori_loop(0, _N_CHUNKS, fire, ())

    def drain(i, _):
        for s in range(_NSEM):
            pltpu.make_async_copy(
                table_hbm.at[0], rows_v.at[0], sems[s]
            ).wait()
        return ()

    lax.fori_loop(0, _B_PER_W // _NSEM, drain, ())
    pltpu.sync_copy(rows_v, out_hbm.at[pl.ds(base, _B_PER_W)])


def kernel(x, table):
    return _emb_lookup(x.astype(jnp.int32), table)
